# Initial kernel scaffold; baseline (speedup 1.0000x reference)
import jax
import jax.numpy as jnp
from jax.experimental import pallas as pl


def kernel(x, pos_table):
    raise NotImplementedError("write your pallas kernel here")


# speedup vs baseline: 1.6688x; 1.6688x over previous
import jax
import jax.numpy as jnp
from jax.experimental import pallas as pl


def _add_pos_kernel(x_ref, pos_ref, o_ref):
    o_ref[...] = x_ref[...] + pos_ref[...][None]


def kernel(x, pos_table):
    batch, seq_len, d_model = x.shape
    block_s = 512
    while seq_len % block_s:
        block_s //= 2

    grid = (seq_len // block_s, batch)
    return pl.pallas_call(
        _add_pos_kernel,
        grid=grid,
        in_specs=[
            pl.BlockSpec((1, block_s, d_model), lambda j, b: (b, j, 0)),
            pl.BlockSpec((block_s, d_model), lambda j, b: (j, 0)),
        ],
        out_specs=pl.BlockSpec((1, block_s, d_model), lambda j, b: (b, j, 0)),
        out_shape=jax.ShapeDtypeStruct(x.shape, x.dtype),
    )(x, pos_table)


# speedup vs baseline: 1.7376x; 1.0412x over previous
import jax
import jax.numpy as jnp
from jax.experimental import pallas as pl


def _add_pos_kernel(x_ref, pos_ref, o_ref):
    o_ref[...] = x_ref[...] + pos_ref[...][None]


def kernel(x, pos_table):
    batch, seq_len, d_model = x.shape
    block_s = 1024
    while seq_len % block_s:
        block_s //= 2

    grid = (seq_len // block_s, batch)
    return pl.pallas_call(
        _add_pos_kernel,
        grid=grid,
        in_specs=[
            pl.BlockSpec((1, block_s, d_model), lambda j, b: (b, j, 0)),
            pl.BlockSpec((block_s, d_model), lambda j, b: (j, 0)),
        ],
        out_specs=pl.BlockSpec((1, block_s, d_model), lambda j, b: (b, j, 0)),
        out_shape=jax.ShapeDtypeStruct(x.shape, x.dtype),
    )(x, pos_table)
